# baseline (speedup 1.0000x reference)
import functools

import jax
import jax.numpy as jnp
from jax import lax
from jax.experimental import pallas as pl
from jax.experimental.pallas import tpu as pltpu
from jax.experimental.pallas import tpu_sc as plsc

N = 4096
LANES = 16
NC = 2
NS = 16
NW = NC * NS
T_TC = 2048
SC_ROWS = N - T_TC
ROWS_PER_W = SC_ROWS // NW
CH = 8
NCHUNK = ROWS_PER_W // CH
BM = 512


def _tc_body(p_ref, a_ref, o_ref):
    o_ref[...] = jnp.sum(a_ref[...], axis=1, keepdims=True) * p_ref[0, 0]


def _sc_body(p_hbm, a_hbm, out_hbm, p_v, buf0, buf1, sums, sem0, sem1):
    wid = lax.axis_index("s") * NC + lax.axis_index("c")
    base = T_TC + wid * ROWS_PER_W

    pltpu.sync_copy(p_hbm, p_v)
    pv = p_v[...][0]
    lane = lax.iota(jnp.int32, LANES)

    bufs = (buf0, buf1)
    sems = (sem0, sem1)
    copies = [None, None]
    copies[0] = pltpu.async_copy(a_hbm.at[pl.ds(base, CH)], buf0, sem0)
    vec = jnp.zeros((LANES,), jnp.float32)
    for g in range(NCHUNK):
        if g + 1 < NCHUNK:
            nb = (g + 1) % 2
            copies[nb] = pltpu.async_copy(
                a_hbm.at[pl.ds(base + (g + 1) * CH, CH)], bufs[nb], sems[nb]
            )
        cb = g % 2
        copies[cb].wait()
        buf = bufs[cb]
        lane_off = (g % 2) * CH

        def row_body(r, vec, buf=buf, lane_off=lane_off):
            def col_body(j, accs):
                a0, a1, a2, a3 = accs
                col = j * 128
                a0 = a0 + buf[r, pl.ds(col, LANES)]
                a1 = a1 + buf[r, pl.ds(col + 16, LANES)]
                a2 = a2 + buf[r, pl.ds(col + 32, LANES)]
                a3 = a3 + buf[r, pl.ds(col + 48, LANES)]
                a0 = a0 + buf[r, pl.ds(col + 64, LANES)]
                a1 = a1 + buf[r, pl.ds(col + 80, LANES)]
                a2 = a2 + buf[r, pl.ds(col + 96, LANES)]
                a3 = a3 + buf[r, pl.ds(col + 112, LANES)]
                return a0, a1, a2, a3

            z = jnp.zeros((LANES,), jnp.float32)
            a0, a1, a2, a3 = lax.fori_loop(0, N // 128, col_body, (z, z, z, z))
            total = (a0 + a1) + (a2 + a3)
            s = jnp.sum(total)
            return jnp.where(lane == r + lane_off, s, vec)

        vec = lax.fori_loop(0, CH, row_body, vec)
        if g % 2 == 1:
            sums[pl.ds((g // 2) * LANES, LANES)] = vec * pv
            vec = jnp.zeros((LANES,), jnp.float32)

    pltpu.sync_copy(sums, out_hbm.at[pl.ds(wid * ROWS_PER_W, ROWS_PER_W)])


@jax.jit
def _hybrid(p2, p16, A):
    mesh = plsc.VectorSubcoreMesh(core_axis_name="c", subcore_axis_name="s")
    sc_k = pl.kernel(
        _sc_body,
        out_type=jax.ShapeDtypeStruct((SC_ROWS,), jnp.float32),
        mesh=mesh,
        compiler_params=pltpu.CompilerParams(needs_layout_passes=False),
        scratch_types=[
            pltpu.VMEM((LANES,), jnp.float32),
            pltpu.VMEM((CH, N), jnp.float32),
            pltpu.VMEM((CH, N), jnp.float32),
            pltpu.VMEM((ROWS_PER_W,), jnp.float32),
            pltpu.SemaphoreType.DMA,
            pltpu.SemaphoreType.DMA,
        ],
    )
    y_sc = sc_k(p16, A)
    y_tc = pl.pallas_call(
        _tc_body,
        grid=(T_TC // BM,),
        in_specs=[
            pl.BlockSpec((1, 1), lambda i: (0, 0), memory_space=pltpu.SMEM),
            pl.BlockSpec((BM, N), lambda i: (i, 0)),
        ],
        out_specs=pl.BlockSpec((BM, 1), lambda i: (i, 0)),
        out_shape=jax.ShapeDtypeStruct((T_TC, 1), jnp.float32),
    )(p2, A)
    return jnp.concatenate([y_tc, y_sc.reshape(SC_ROWS, 1)], axis=0)


def kernel(p, A):
    return _hybrid(p.reshape(1, 1), jnp.broadcast_to(p, (LANES,)), A)
